# baseline (speedup 1.0000x reference)
import jax
import jax.numpy as jnp
from jax.experimental import pallas as pl

_TBLK = 16


def _mulsum_body(x_ref, coef_ref, o_ref):
    o_ref[...] = jnp.sum(x_ref[...] * coef_ref[...][None], axis=-1)


def kernel(x, coef):
    num_trips, num_items, num_params = x.shape
    grid = (num_trips // _TBLK,)
    return pl.pallas_call(
        _mulsum_body,
        grid=grid,
        in_specs=[
            pl.BlockSpec((_TBLK, num_items, num_params), lambda i: (i, 0, 0)),
            pl.BlockSpec((num_items, num_params), lambda i: (0, 0)),
        ],
        out_specs=pl.BlockSpec((_TBLK, num_items), lambda i: (i, 0)),
        out_shape=jax.ShapeDtypeStruct((num_trips, num_items), jnp.float32),
    )(x, coef)
